# baseline (speedup 1.0000x reference)
import jax
import jax.numpy as jnp
from jax import lax
from jax.experimental import pallas as pl
from jax.experimental.pallas import tpu as pltpu

_H = 256
_W = 256
_K = 18
_BB = 2


def _box_kernel(xr_ref, xc_ref, out_ref):
    bi = pl.program_id(0)
    ri = lax.broadcasted_iota(jnp.int32, (_H, 1), 0)
    cj = lax.broadcasted_iota(jnp.int32, (1, _W), 1)
    for sub in range(_BB):
        for ki in range(_K):
            r0 = xr_ref[bi * _BB + sub, ki]
            c0 = xc_ref[bi * _BB + sub, ki]
            rowf = jnp.where((ri - r0).astype(jnp.uint32) <= 10,
                             jnp.float32(1.0), jnp.float32(0.0))
            colf = jnp.where((cj - c0).astype(jnp.uint32) <= 10,
                             jnp.float32(1.0), jnp.float32(0.0))
            out_ref[sub, ki] = rowf * colf


def kernel(x):
    b, k, _ = x.shape
    xr = x[:, :, 0] - 5
    xc = x[:, :, 1] - 5
    grid_spec = pltpu.PrefetchScalarGridSpec(
        num_scalar_prefetch=2,
        grid=(b // _BB,),
        in_specs=[],
        out_specs=pl.BlockSpec((_BB, _K, _H, _W),
                               lambda bi, xr_s, xc_s: (bi, 0, 0, 0)),
    )
    y = pl.pallas_call(
        _box_kernel,
        grid_spec=grid_spec,
        out_shape=jax.ShapeDtypeStruct((b, k, _H, _W), jnp.float32),
    )(xr, xc)
    return jnp.transpose(y, (0, 2, 3, 1))
